# baseline (speedup 1.0000x reference)
import functools

import jax
import jax.numpy as jnp
from jax.experimental import pallas as pl
from jax.experimental.pallas import tpu as pltpu

_BN_EPS = 1e-5


def _sq_dist(src, dst):
    d = -2.0 * jnp.matmul(src, jnp.transpose(dst, (0, 2, 1)))
    d = d + jnp.sum(src ** 2, -1)[..., None]
    d = d + jnp.sum(dst ** 2, -1)[:, None, :]
    return d


def _onehot_rows(points, idx):
    N = points.shape[1]
    oh = (idx[..., None] == jnp.arange(N, dtype=idx.dtype))
    return jnp.sum(jnp.where(oh[..., None], points[:, None, :, :], 0.0), axis=2)


def _fps(xyz, npoint):
    B, N, _ = xyz.shape

    def body(i, state):
        centroids, distance, farthest = state
        centroids = centroids.at[:, i].set(farthest)
        onehot = (jax.lax.broadcasted_iota(jnp.int32, (B, N), 1)
                  == farthest[:, None]).astype(xyz.dtype)
        centroid = jnp.sum(xyz * onehot[..., None], axis=1, keepdims=True)
        dist = jnp.sum((xyz - centroid) ** 2, -1)
        distance = jnp.minimum(distance, dist)
        farthest = jnp.argmax(distance, axis=-1).astype(jnp.int32)
        return centroids, distance, farthest

    centroids = jnp.zeros((B, npoint), dtype=jnp.int32)
    distance = jnp.full((B, N), 1e10, dtype=xyz.dtype)
    farthest = jnp.zeros((B,), dtype=jnp.int32)
    centroids, _, _ = jax.lax.fori_loop(0, npoint, body,
                                        (centroids, distance, farthest), unroll=8)
    return centroids


def _fps_kern(xyz_ref, out_ref, *, npoint, n):
    x0 = xyz_ref[0, 0]
    x1 = xyz_ref[0, 1]
    x2 = xyz_ref[0, 2]
    bb = x0.shape[1]
    n_iota = jax.lax.broadcasted_iota(jnp.int32, (n, bb), 0)

    def body(i, state):
        distance, farv = state
        out_ref[0, pl.ds(i, 1), :] = farv
        ohf = jnp.where(n_iota == farv, 1.0, 0.0)
        c0 = jnp.sum(x0 * ohf, axis=0, keepdims=True)
        c1 = jnp.sum(x1 * ohf, axis=0, keepdims=True)
        c2 = jnp.sum(x2 * ohf, axis=0, keepdims=True)
        d = (x0 - c0) ** 2 + (x1 - c1) ** 2 + (x2 - c2) ** 2
        distance = jnp.minimum(distance, d)
        m = jnp.max(distance, axis=0, keepdims=True)
        cand = jnp.where(distance == m, n_iota, n)
        farv = jnp.min(cand, axis=0, keepdims=True)
        return distance, farv

    dist0 = jnp.full((n, bb), 1e10, jnp.float32)
    far0 = jnp.zeros((1, bb), jnp.int32)
    jax.lax.fori_loop(0, npoint, body, (dist0, far0))


def _fps_pallas(xyz, npoint):
    B, N, _ = xyz.shape
    halves = 2
    bb = B // halves
    xt = jnp.transpose(xyz, (2, 1, 0))
    xt = jnp.transpose(xt.reshape(3, N, halves, bb), (2, 0, 1, 3))
    out = pl.pallas_call(
        functools.partial(_fps_kern, npoint=npoint, n=N),
        grid=(halves,),
        in_specs=[pl.BlockSpec((1, 3, N, bb), lambda i: (i, 0, 0, 0))],
        out_specs=pl.BlockSpec((1, npoint, bb), lambda i: (i, 0, 0)),
        out_shape=jax.ShapeDtypeStruct((halves, npoint, bb), jnp.int32),
        compiler_params=pltpu.CompilerParams(dimension_semantics=("parallel",),
                                             vmem_limit_bytes=48 * 1024 * 1024),
    )(xt)
    return out.transpose(0, 2, 1).reshape(B, npoint)


def _sel_gather_kern(sqdt_ref, trow_ref, src_ref, nxp_ref, ltt_ref, e16_ref,
                     w0_ref, x0_ref, st_ref, *, chunks, chunk, kns, r2):
    srcb = src_ref[0]
    nxb = nxp_ref[0]
    sq = sqdt_ref[0]
    gpc = chunk // kns
    validb = jnp.where(sq <= r2, 1.0, 0.0).astype(jnp.bfloat16)
    rank = jax.lax.dot_general(ltt_ref[...], validb, (((0,), (0,)), ((), ())),
                               preferred_element_type=jnp.float32)
    rv = jnp.where(sq <= r2, jnp.minimum(rank, float(kns + 1)), 0.0)
    rvb = rv.astype(jnp.bfloat16)
    e16 = e16_ref[...]
    w0 = w0_ref[...]
    s1 = jnp.zeros((1, w0.shape[1]), jnp.float32)
    s2 = jnp.zeros((1, w0.shape[1]), jnp.float32)
    for cc in range(chunks):
        sl = slice(gpc * cc, gpc * (cc + 1))
        rvr = jax.lax.dot_general(rvb[:, sl], e16, (((1,), (0,)), ((), ())),
                                  preferred_element_type=jnp.float32)
        tr = trow_ref[0, cc:cc + 1, :]
        oht = jnp.where(rvr == tr, 1.0, 0.0).astype(jnp.bfloat16)
        g = jax.lax.dot_general(oht, srcb, (((0,), (0,)), ((), ())),
                                preferred_element_type=jnp.float32)
        nxg = nxb[sl][:, None, :]
        ctr = jnp.broadcast_to(nxg, (gpc, kns, nxb.shape[-1])).reshape(chunk, -1)
        x0c = (g - ctr).astype(jnp.bfloat16)
        x0_ref[pl.ds(cc * chunk, chunk), :] = x0c
        y0 = jnp.dot(x0c, w0, preferred_element_type=jnp.float32)
        s1 = s1 + jnp.sum(y0, axis=0, keepdims=True)
        s2 = s2 + jnp.sum(y0 * y0, axis=0, keepdims=True)
    st_ref[...] = jnp.concatenate(
        [s1, s2, jnp.zeros((6, s1.shape[1]), jnp.float32)], axis=0)


def _sel_gather(sqdt, trow, src, nxp, w0p, *, nsample):
    B, N, C = src.shape
    S = nxp.shape[1]
    chunks, chunk = trow.shape[1], trow.shape[2]
    rows = chunks * chunk
    c1 = w0p.shape[1]
    ltt = (jnp.arange(N)[:, None] <= jnp.arange(N)[None, :]).astype(jnp.bfloat16)
    e16 = (jnp.arange(chunk // nsample)[:, None]
           == (jnp.arange(chunk)[None, :] // nsample)).astype(jnp.bfloat16)
    r2 = float(0.4 ** 2)
    return pl.pallas_call(
        functools.partial(_sel_gather_kern, chunks=chunks, chunk=chunk,
                          kns=nsample, r2=r2),
        grid=(B,),
        in_specs=[pl.BlockSpec((1, N, S), lambda i: (i, 0, 0)),
                  pl.BlockSpec((1, chunks, chunk), lambda i: (i, 0, 0)),
                  pl.BlockSpec((1, N, C), lambda i: (i, 0, 0)),
                  pl.BlockSpec((1, S, C), lambda i: (i, 0, 0)),
                  pl.BlockSpec((N, N), lambda i: (0, 0)),
                  pl.BlockSpec((chunk // nsample, chunk), lambda i: (0, 0)),
                  pl.BlockSpec((C, c1), lambda i: (0, 0))],
        out_specs=(pl.BlockSpec((rows, C), lambda i: (i, 0)),
                   pl.BlockSpec((8, c1), lambda i: (i, 0))),
        out_shape=(jax.ShapeDtypeStruct((B * rows, C), jnp.bfloat16),
                   jax.ShapeDtypeStruct((B * 8, c1), jnp.float32)),
        compiler_params=pltpu.CompilerParams(dimension_semantics=("parallel",),
                                             vmem_limit_bytes=48 * 1024 * 1024),
    )(sqdt, trow, src, nxp, ltt, e16, w0p)


def _colstats(y):
    s1 = jnp.sum(y, axis=0, keepdims=True)
    s2 = jnp.sum(y * y, axis=0, keepdims=True)
    return jnp.concatenate([s1, s2, jnp.zeros((6, y.shape[1]), jnp.float32)], axis=0)


def _bn_mm_stats_kern(x_ref, w_ref, sc_ref, sh_ref, wn_ref, xn_ref, st_ref):
    y = jnp.dot(x_ref[...], w_ref[...], preferred_element_type=jnp.float32)
    xn = jnp.maximum(y * sc_ref[...] + sh_ref[...], 0.0).astype(jnp.bfloat16)
    xn_ref[...] = xn
    yn = jnp.dot(xn, wn_ref[...], preferred_element_type=jnp.float32)
    st_ref[...] = _colstats(yn)


def _bn_maxpool_kern(x_ref, w_ref, sc_ref, sh_ref, o_ref, *, gpt, nsample):
    y = jnp.dot(x_ref[...], w_ref[...], preferred_element_type=jnp.float32)
    z = jnp.maximum(y * sc_ref[...] + sh_ref[...], 0.0)
    o_ref[...] = jnp.max(z.reshape(gpt, nsample, z.shape[-1]), axis=1)


def _fold_bn(stats, gamma, beta, p, axis_name=None):
    part = stats.reshape(-1, 8, stats.shape[-1]).sum(axis=0)
    if axis_name is not None:
        part = jax.lax.psum(part, axis_name)
    mean = part[0] / float(p)
    var = jnp.maximum(part[1] / float(p) - mean * mean, 0.0)
    sc = gamma.reshape(-1).astype(jnp.float32) * jax.lax.rsqrt(var + _BN_EPS)
    sh = beta.reshape(-1).astype(jnp.float32) - mean * sc
    return sc.reshape(1, -1), sh.reshape(1, -1)


def _mlp_pool(x0, stats0, params, *, nsample, p_total, axis_name=None):
    P, c0 = x0.shape
    (w0, g0, b0), (w1, g1, b1), (w2, g2, b2) = params
    c1, c2, c3 = w0.shape[1], w1.shape[1], w2.shape[1]
    tile = 8192
    nt = P // tile
    gpt = tile // nsample
    cp = pltpu.CompilerParams(dimension_semantics=("parallel",),
                              vmem_limit_bytes=48 * 1024 * 1024)

    sc0, sh0 = _fold_bn(stats0, g0, b0, p_total, axis_name)

    x1, stats1 = pl.pallas_call(
        _bn_mm_stats_kern, grid=(nt,),
        in_specs=[pl.BlockSpec((tile, c0), lambda i: (i, 0)),
                  pl.BlockSpec((c0, c1), lambda i: (0, 0)),
                  pl.BlockSpec((1, c1), lambda i: (0, 0)),
                  pl.BlockSpec((1, c1), lambda i: (0, 0)),
                  pl.BlockSpec((c1, c2), lambda i: (0, 0))],
        out_specs=(pl.BlockSpec((tile, c1), lambda i: (i, 0)),
                   pl.BlockSpec((8, c2), lambda i: (i, 0))),
        out_shape=(jax.ShapeDtypeStruct((P, c1), jnp.bfloat16),
                   jax.ShapeDtypeStruct((nt * 8, c2), jnp.float32)),
        compiler_params=cp,
    )(x0, w0, sc0, sh0, w1)
    sc1, sh1 = _fold_bn(stats1, g1, b1, p_total, axis_name)

    x2, stats2 = pl.pallas_call(
        _bn_mm_stats_kern, grid=(nt,),
        in_specs=[pl.BlockSpec((tile, c1), lambda i: (i, 0)),
                  pl.BlockSpec((c1, c2), lambda i: (0, 0)),
                  pl.BlockSpec((1, c2), lambda i: (0, 0)),
                  pl.BlockSpec((1, c2), lambda i: (0, 0)),
                  pl.BlockSpec((c2, c3), lambda i: (0, 0))],
        out_specs=(pl.BlockSpec((tile, c2), lambda i: (i, 0)),
                   pl.BlockSpec((8, c3), lambda i: (i, 0))),
        out_shape=(jax.ShapeDtypeStruct((P, c2), jnp.bfloat16),
                   jax.ShapeDtypeStruct((nt * 8, c3), jnp.float32)),
        compiler_params=cp,
    )(x1, w1, sc1, sh1, w2)
    sc2, sh2 = _fold_bn(stats2, g2, b2, p_total, axis_name)

    out = pl.pallas_call(
        functools.partial(_bn_maxpool_kern, gpt=gpt, nsample=nsample),
        grid=(nt,),
        in_specs=[pl.BlockSpec((tile, c2), lambda i: (i, 0)),
                  pl.BlockSpec((c2, c3), lambda i: (0, 0)),
                  pl.BlockSpec((1, c3), lambda i: (0, 0)),
                  pl.BlockSpec((1, c3), lambda i: (0, 0))],
        out_specs=pl.BlockSpec((gpt, c3), lambda i: (i, 0)),
        out_shape=jax.ShapeDtypeStruct((P // nsample, c3), jnp.float32),
        compiler_params=cp,
    )(x2, w2, sc2, sh2)
    return out


def _kernel_body(xyz, points, w0, gamma0, beta0, w1, gamma1, beta1,
                 w2, gamma2, beta2, *, p_total, axis_name):
    npoint, radius, nsample = 128, 0.4, 64
    B, _, N = xyz.shape
    xyz_t = jnp.transpose(xyz, (0, 2, 1))
    pts_t = jnp.transpose(points, (0, 2, 1))

    fps_idx = _fps(xyz_t, npoint)
    new_xyz = _onehot_rows(xyz_t, fps_idx)

    sqrdists = _sq_dist(new_xyz, xyz_t)
    sqdt = jnp.transpose(sqrdists, (0, 2, 1))
    cnt = jnp.sum((sqrdists <= radius ** 2).astype(jnp.int32), axis=-1)
    k_iota = jnp.arange(nsample, dtype=jnp.int32)
    trow = jnp.where(k_iota[None, None, :] < cnt[:, :, None], k_iota + 1, 1)
    trow = trow.reshape(B, 8, (npoint * nsample) // 8).astype(jnp.float32)

    c_in = 3 + pts_t.shape[-1]
    c_pad = ((c_in + 15) // 16) * 16
    src = jnp.concatenate(
        [xyz_t, pts_t, jnp.zeros((B, N, c_pad - c_in), jnp.float32)],
        axis=-1).astype(jnp.bfloat16)
    nxp = jnp.concatenate(
        [new_xyz, jnp.zeros((B, npoint, c_pad - 3), jnp.float32)], axis=-1)

    w0p = jnp.pad(w0, ((0, c_pad - c_in), (0, 0))).astype(jnp.bfloat16)
    x0, stats0 = _sel_gather(sqdt, trow, src, nxp, w0p, nsample=nsample)

    params = [(w0p, gamma0, beta0),
              (w1.astype(jnp.bfloat16), gamma1, beta1),
              (w2.astype(jnp.bfloat16), gamma2, beta2)]
    out = _mlp_pool(x0, stats0, params, nsample=nsample,
                    p_total=p_total, axis_name=axis_name)

    feat = out.reshape(B, npoint, -1).transpose(0, 2, 1)
    return jnp.transpose(new_xyz, (0, 2, 1)), feat


def kernel(xyz, points, w0, gamma0, beta0, w1, gamma1, beta1, w2, gamma2, beta2):
    npoint, nsample = 128, 64
    B = xyz.shape[0]
    p_total = B * npoint * nsample
    devs = jax.devices()
    nd = 2 if (len(devs) >= 2 and B % 2 == 0) else 1
    if nd == 1:
        return _kernel_body(xyz, points, w0, gamma0, beta0, w1, gamma1, beta1,
                            w2, gamma2, beta2, p_total=p_total, axis_name=None)
    import numpy as _np
    from jax.sharding import Mesh, PartitionSpec as P
    from jax.experimental.shard_map import shard_map
    mesh = Mesh(_np.array(devs[:nd]), ("b",))
    body = functools.partial(_kernel_body, p_total=p_total, axis_name="b")
    f = shard_map(
        body, mesh=mesh,
        in_specs=(P("b"), P("b"), P(), P(), P(), P(), P(), P(), P(), P(), P()),
        out_specs=(P("b"), P("b")),
        check_rep=False,
    )
    return f(xyz, points, w0, gamma0, beta0, w1, gamma1, beta1, w2, gamma2, beta2)


# speedup vs baseline: 1.0399x; 1.0399x over previous
import functools

import jax
import jax.numpy as jnp
from jax.experimental import pallas as pl
from jax.experimental.pallas import tpu as pltpu

_BN_EPS = 1e-5


def _sq_dist(src, dst):
    d = -2.0 * jnp.matmul(src, jnp.transpose(dst, (0, 2, 1)))
    d = d + jnp.sum(src ** 2, -1)[..., None]
    d = d + jnp.sum(dst ** 2, -1)[:, None, :]
    return d


def _onehot_rows(points, idx):
    N = points.shape[1]
    oh = (idx[..., None] == jnp.arange(N, dtype=idx.dtype))
    return jnp.sum(jnp.where(oh[..., None], points[:, None, :, :], 0.0), axis=2)


def _fps(xyz, npoint):
    B, N, _ = xyz.shape

    def body(i, state):
        centroids, distance, farthest = state
        centroids = centroids.at[:, i].set(farthest)
        onehot = (jax.lax.broadcasted_iota(jnp.int32, (B, N), 1)
                  == farthest[:, None]).astype(xyz.dtype)
        centroid = jnp.sum(xyz * onehot[..., None], axis=1, keepdims=True)
        dist = jnp.sum((xyz - centroid) ** 2, -1)
        distance = jnp.minimum(distance, dist)
        farthest = jnp.argmax(distance, axis=-1).astype(jnp.int32)
        return centroids, distance, farthest

    centroids = jnp.zeros((B, npoint), dtype=jnp.int32)
    distance = jnp.full((B, N), 1e10, dtype=xyz.dtype)
    farthest = jnp.zeros((B,), dtype=jnp.int32)
    centroids, _, _ = jax.lax.fori_loop(0, npoint, body,
                                        (centroids, distance, farthest), unroll=8)
    return centroids


def _fps_kern(xyz_ref, out_ref, *, npoint, n):
    x0 = xyz_ref[0, 0]
    x1 = xyz_ref[0, 1]
    x2 = xyz_ref[0, 2]
    bb = x0.shape[1]
    n_iota = jax.lax.broadcasted_iota(jnp.int32, (n, bb), 0)

    def body(i, state):
        distance, farv = state
        out_ref[0, pl.ds(i, 1), :] = farv
        ohf = jnp.where(n_iota == farv, 1.0, 0.0)
        c0 = jnp.sum(x0 * ohf, axis=0, keepdims=True)
        c1 = jnp.sum(x1 * ohf, axis=0, keepdims=True)
        c2 = jnp.sum(x2 * ohf, axis=0, keepdims=True)
        d = (x0 - c0) ** 2 + (x1 - c1) ** 2 + (x2 - c2) ** 2
        distance = jnp.minimum(distance, d)
        m = jnp.max(distance, axis=0, keepdims=True)
        cand = jnp.where(distance == m, n_iota, n)
        farv = jnp.min(cand, axis=0, keepdims=True)
        return distance, farv

    dist0 = jnp.full((n, bb), 1e10, jnp.float32)
    far0 = jnp.zeros((1, bb), jnp.int32)
    jax.lax.fori_loop(0, npoint, body, (dist0, far0))


def _fps_pallas(xyz, npoint):
    B, N, _ = xyz.shape
    halves = 2
    bb = B // halves
    xt = jnp.transpose(xyz, (2, 1, 0))
    xt = jnp.transpose(xt.reshape(3, N, halves, bb), (2, 0, 1, 3))
    out = pl.pallas_call(
        functools.partial(_fps_kern, npoint=npoint, n=N),
        grid=(halves,),
        in_specs=[pl.BlockSpec((1, 3, N, bb), lambda i: (i, 0, 0, 0))],
        out_specs=pl.BlockSpec((1, npoint, bb), lambda i: (i, 0, 0)),
        out_shape=jax.ShapeDtypeStruct((halves, npoint, bb), jnp.int32),
        compiler_params=pltpu.CompilerParams(dimension_semantics=("parallel",),
                                             vmem_limit_bytes=48 * 1024 * 1024),
    )(xt)
    return out.transpose(0, 2, 1).reshape(B, npoint)


def _sel_gather_kern(sqdt_ref, trow_ref, src_ref, nxp_ref, ltt_ref, e16_ref,
                     w0_ref, x0_ref, st_ref, *, chunks, chunk, kns, r2):
    srcb = src_ref[0]
    nxb = nxp_ref[0]
    sq = sqdt_ref[0]
    gpc = chunk // kns
    validb = jnp.where(sq <= r2, 1.0, 0.0).astype(jnp.bfloat16)
    rank = jax.lax.dot_general(ltt_ref[...], validb, (((0,), (0,)), ((), ())),
                               preferred_element_type=jnp.float32)
    rv = jnp.where(sq <= r2, jnp.minimum(rank, float(kns + 1)), 0.0)
    rvb = rv.astype(jnp.bfloat16)
    e16 = e16_ref[...]
    w0 = w0_ref[...]
    s1 = jnp.zeros((1, w0.shape[1]), jnp.float32)
    s2 = jnp.zeros((1, w0.shape[1]), jnp.float32)
    for cc in range(chunks):
        sl = slice(gpc * cc, gpc * (cc + 1))
        rvr = jax.lax.dot_general(rvb[:, sl], e16, (((1,), (0,)), ((), ())),
                                  preferred_element_type=jnp.float32)
        tr = trow_ref[0, cc:cc + 1, :]
        oht = jnp.where(rvr == tr, 1.0, 0.0).astype(jnp.bfloat16)
        g = jax.lax.dot_general(oht, srcb, (((0,), (0,)), ((), ())),
                                preferred_element_type=jnp.float32)
        nxg = nxb[sl][:, None, :]
        ctr = jnp.broadcast_to(nxg, (gpc, kns, nxb.shape[-1])).reshape(chunk, -1)
        x0c = (g - ctr).astype(jnp.bfloat16)
        x0_ref[pl.ds(cc * chunk, chunk), :] = x0c
        y0 = jnp.dot(x0c, w0, preferred_element_type=jnp.float32)
        s1 = s1 + jnp.sum(y0, axis=0, keepdims=True)
        s2 = s2 + jnp.sum(y0 * y0, axis=0, keepdims=True)
    st_ref[...] = jnp.concatenate(
        [s1, s2, jnp.zeros((6, s1.shape[1]), jnp.float32)], axis=0)


def _sel_gather(sqdt, trow, src, nxp, w0p, *, nsample):
    B, N, C = src.shape
    S = nxp.shape[1]
    chunks, chunk = trow.shape[1], trow.shape[2]
    rows = chunks * chunk
    c1 = w0p.shape[1]
    ltt = (jnp.arange(N)[:, None] <= jnp.arange(N)[None, :]).astype(jnp.bfloat16)
    e16 = (jnp.arange(chunk // nsample)[:, None]
           == (jnp.arange(chunk)[None, :] // nsample)).astype(jnp.bfloat16)
    r2 = float(0.4 ** 2)
    return pl.pallas_call(
        functools.partial(_sel_gather_kern, chunks=chunks, chunk=chunk,
                          kns=nsample, r2=r2),
        grid=(B,),
        in_specs=[pl.BlockSpec((1, N, S), lambda i: (i, 0, 0)),
                  pl.BlockSpec((1, chunks, chunk), lambda i: (i, 0, 0)),
                  pl.BlockSpec((1, N, C), lambda i: (i, 0, 0)),
                  pl.BlockSpec((1, S, C), lambda i: (i, 0, 0)),
                  pl.BlockSpec((N, N), lambda i: (0, 0)),
                  pl.BlockSpec((chunk // nsample, chunk), lambda i: (0, 0)),
                  pl.BlockSpec((C, c1), lambda i: (0, 0))],
        out_specs=(pl.BlockSpec((rows, C), lambda i: (i, 0)),
                   pl.BlockSpec((8, c1), lambda i: (i, 0))),
        out_shape=(jax.ShapeDtypeStruct((B * rows, C), jnp.bfloat16),
                   jax.ShapeDtypeStruct((B * 8, c1), jnp.float32)),
        compiler_params=pltpu.CompilerParams(dimension_semantics=("parallel",),
                                             vmem_limit_bytes=48 * 1024 * 1024),
    )(sqdt, trow, src, nxp, ltt, e16, w0p)


def _colstats(y):
    s1 = jnp.sum(y, axis=0, keepdims=True)
    s2 = jnp.sum(y * y, axis=0, keepdims=True)
    return jnp.concatenate([s1, s2, jnp.zeros((6, y.shape[1]), jnp.float32)], axis=0)


def _bn_mm_stats_kern(x_ref, w_ref, sc_ref, sh_ref, wn_ref, xn_ref, st_ref):
    y = jnp.dot(x_ref[...], w_ref[...], preferred_element_type=jnp.float32)
    xn = jnp.maximum(y * sc_ref[...] + sh_ref[...], 0.0).astype(jnp.bfloat16)
    xn_ref[...] = xn
    yn = jnp.dot(xn, wn_ref[...], preferred_element_type=jnp.float32)
    st_ref[...] = _colstats(yn)


def _bn_maxpool_kern(x_ref, w_ref, sc_ref, sh_ref, o_ref, *, gpt, nsample):
    y = jnp.dot(x_ref[...], w_ref[...], preferred_element_type=jnp.float32)
    z = jnp.maximum(y * sc_ref[...] + sh_ref[...], 0.0)
    o_ref[...] = jnp.max(z.reshape(gpt, nsample, z.shape[-1]), axis=1)


def _fold_bn(stats, gamma, beta, p, axis_name=None):
    part = stats.reshape(-1, 8, stats.shape[-1]).sum(axis=0)
    if axis_name is not None:
        part = jax.lax.psum(part, axis_name)
    mean = part[0] / float(p)
    var = jnp.maximum(part[1] / float(p) - mean * mean, 0.0)
    sc = gamma.reshape(-1).astype(jnp.float32) * jax.lax.rsqrt(var + _BN_EPS)
    sh = beta.reshape(-1).astype(jnp.float32) - mean * sc
    return sc.reshape(1, -1), sh.reshape(1, -1)


def _mlp_pool(x0, stats0, params, *, nsample, p_total, axis_name=None):
    P, c0 = x0.shape
    (w0, g0, b0), (w1, g1, b1), (w2, g2, b2) = params
    c1, c2, c3 = w0.shape[1], w1.shape[1], w2.shape[1]
    tile = 8192
    nt = P // tile
    gpt = tile // nsample
    cp = pltpu.CompilerParams(dimension_semantics=("parallel",),
                              vmem_limit_bytes=48 * 1024 * 1024)

    sc0, sh0 = _fold_bn(stats0, g0, b0, p_total, axis_name)

    x1, stats1 = pl.pallas_call(
        _bn_mm_stats_kern, grid=(nt,),
        in_specs=[pl.BlockSpec((tile, c0), lambda i: (i, 0)),
                  pl.BlockSpec((c0, c1), lambda i: (0, 0)),
                  pl.BlockSpec((1, c1), lambda i: (0, 0)),
                  pl.BlockSpec((1, c1), lambda i: (0, 0)),
                  pl.BlockSpec((c1, c2), lambda i: (0, 0))],
        out_specs=(pl.BlockSpec((tile, c1), lambda i: (i, 0)),
                   pl.BlockSpec((8, c2), lambda i: (i, 0))),
        out_shape=(jax.ShapeDtypeStruct((P, c1), jnp.bfloat16),
                   jax.ShapeDtypeStruct((nt * 8, c2), jnp.float32)),
        compiler_params=cp,
    )(x0, w0, sc0, sh0, w1)
    sc1, sh1 = _fold_bn(stats1, g1, b1, p_total, axis_name)

    x2, stats2 = pl.pallas_call(
        _bn_mm_stats_kern, grid=(nt,),
        in_specs=[pl.BlockSpec((tile, c1), lambda i: (i, 0)),
                  pl.BlockSpec((c1, c2), lambda i: (0, 0)),
                  pl.BlockSpec((1, c2), lambda i: (0, 0)),
                  pl.BlockSpec((1, c2), lambda i: (0, 0)),
                  pl.BlockSpec((c2, c3), lambda i: (0, 0))],
        out_specs=(pl.BlockSpec((tile, c2), lambda i: (i, 0)),
                   pl.BlockSpec((8, c3), lambda i: (i, 0))),
        out_shape=(jax.ShapeDtypeStruct((P, c2), jnp.bfloat16),
                   jax.ShapeDtypeStruct((nt * 8, c3), jnp.float32)),
        compiler_params=cp,
    )(x1, w1, sc1, sh1, w2)
    sc2, sh2 = _fold_bn(stats2, g2, b2, p_total, axis_name)

    out = pl.pallas_call(
        functools.partial(_bn_maxpool_kern, gpt=gpt, nsample=nsample),
        grid=(nt,),
        in_specs=[pl.BlockSpec((tile, c2), lambda i: (i, 0)),
                  pl.BlockSpec((c2, c3), lambda i: (0, 0)),
                  pl.BlockSpec((1, c3), lambda i: (0, 0)),
                  pl.BlockSpec((1, c3), lambda i: (0, 0))],
        out_specs=pl.BlockSpec((gpt, c3), lambda i: (i, 0)),
        out_shape=jax.ShapeDtypeStruct((P // nsample, c3), jnp.float32),
        compiler_params=cp,
    )(x2, w2, sc2, sh2)
    return out


def _kernel_body(xyz, points, w0, gamma0, beta0, w1, gamma1, beta1,
                 w2, gamma2, beta2, *, p_total, axis_name):
    npoint, radius, nsample = 128, 0.4, 64
    B, _, N = xyz.shape
    xyz_t = jnp.transpose(xyz, (0, 2, 1))
    pts_t = jnp.transpose(points, (0, 2, 1))

    fps_idx = _fps(xyz_t, npoint)
    new_xyz = _onehot_rows(xyz_t, fps_idx)

    sqrdists = _sq_dist(new_xyz, xyz_t)
    sqdt = jnp.transpose(sqrdists, (0, 2, 1))
    cnt = jnp.sum((sqrdists <= radius ** 2).astype(jnp.int32), axis=-1)
    k_iota = jnp.arange(nsample, dtype=jnp.int32)
    trow = jnp.where(k_iota[None, None, :] < cnt[:, :, None], k_iota + 1, 1)
    trow = trow.reshape(B, 8, (npoint * nsample) // 8).astype(jnp.float32)

    c_in = 3 + pts_t.shape[-1]
    c_pad = ((c_in + 15) // 16) * 16
    src = jnp.concatenate(
        [xyz_t, pts_t, jnp.zeros((B, N, c_pad - c_in), jnp.float32)],
        axis=-1).astype(jnp.bfloat16)
    nxp = jnp.concatenate(
        [new_xyz, jnp.zeros((B, npoint, c_pad - 3), jnp.float32)], axis=-1)

    w0p = jnp.pad(w0, ((0, c_pad - c_in), (0, 0))).astype(jnp.bfloat16)
    x0, stats0 = _sel_gather(sqdt, trow, src, nxp, w0p, nsample=nsample)

    params = [(w0p, gamma0, beta0),
              (w1.astype(jnp.bfloat16), gamma1, beta1),
              (w2.astype(jnp.bfloat16), gamma2, beta2)]
    out = _mlp_pool(x0, stats0, params, nsample=nsample,
                    p_total=p_total, axis_name=axis_name)

    feat = out.reshape(B, npoint, -1).transpose(0, 2, 1)
    return jnp.transpose(new_xyz, (0, 2, 1)), feat


def kernel(xyz, points, w0, gamma0, beta0, w1, gamma1, beta1, w2, gamma2, beta2):
    npoint, nsample = 128, 64
    B = xyz.shape[0]
    p_total = B * npoint * nsample
    return _kernel_body(xyz, points, w0, gamma0, beta0, w1, gamma1, beta1,
                        w2, gamma2, beta2, p_total=p_total, axis_name=None)


# speedup vs baseline: 1.0407x; 1.0008x over previous
import functools

import jax
import jax.numpy as jnp
from jax.experimental import pallas as pl
from jax.experimental.pallas import tpu as pltpu

_BN_EPS = 1e-5


def _sq_dist(src, dst):
    d = -2.0 * jnp.matmul(src, jnp.transpose(dst, (0, 2, 1)))
    d = d + jnp.sum(src ** 2, -1)[..., None]
    d = d + jnp.sum(dst ** 2, -1)[:, None, :]
    return d


def _onehot_rows(points, idx):
    N = points.shape[1]
    oh = (idx[..., None] == jnp.arange(N, dtype=idx.dtype))
    return jnp.sum(jnp.where(oh[..., None], points[:, None, :, :], 0.0), axis=2)


def _fps(xyz, npoint):
    B, N, _ = xyz.shape

    def body(i, state):
        centroids, distance, farthest = state
        centroids = centroids.at[:, i].set(farthest)
        onehot = (jax.lax.broadcasted_iota(jnp.int32, (B, N), 1)
                  == farthest[:, None]).astype(xyz.dtype)
        centroid = jnp.sum(xyz * onehot[..., None], axis=1, keepdims=True)
        dist = jnp.sum((xyz - centroid) ** 2, -1)
        distance = jnp.minimum(distance, dist)
        farthest = jnp.argmax(distance, axis=-1).astype(jnp.int32)
        return centroids, distance, farthest

    centroids = jnp.zeros((B, npoint), dtype=jnp.int32)
    distance = jnp.full((B, N), 1e10, dtype=xyz.dtype)
    farthest = jnp.zeros((B,), dtype=jnp.int32)
    centroids, _, _ = jax.lax.fori_loop(0, npoint, body,
                                        (centroids, distance, farthest), unroll=16)
    return centroids


def _fps_kern(xyz_ref, out_ref, *, npoint, n):
    x0 = xyz_ref[0, 0]
    x1 = xyz_ref[0, 1]
    x2 = xyz_ref[0, 2]
    bb = x0.shape[1]
    n_iota = jax.lax.broadcasted_iota(jnp.int32, (n, bb), 0)

    def body(i, state):
        distance, farv = state
        out_ref[0, pl.ds(i, 1), :] = farv
        ohf = jnp.where(n_iota == farv, 1.0, 0.0)
        c0 = jnp.sum(x0 * ohf, axis=0, keepdims=True)
        c1 = jnp.sum(x1 * ohf, axis=0, keepdims=True)
        c2 = jnp.sum(x2 * ohf, axis=0, keepdims=True)
        d = (x0 - c0) ** 2 + (x1 - c1) ** 2 + (x2 - c2) ** 2
        distance = jnp.minimum(distance, d)
        m = jnp.max(distance, axis=0, keepdims=True)
        cand = jnp.where(distance == m, n_iota, n)
        farv = jnp.min(cand, axis=0, keepdims=True)
        return distance, farv

    dist0 = jnp.full((n, bb), 1e10, jnp.float32)
    far0 = jnp.zeros((1, bb), jnp.int32)
    jax.lax.fori_loop(0, npoint, body, (dist0, far0))


def _fps_pallas(xyz, npoint):
    B, N, _ = xyz.shape
    halves = 2
    bb = B // halves
    xt = jnp.transpose(xyz, (2, 1, 0))
    xt = jnp.transpose(xt.reshape(3, N, halves, bb), (2, 0, 1, 3))
    out = pl.pallas_call(
        functools.partial(_fps_kern, npoint=npoint, n=N),
        grid=(halves,),
        in_specs=[pl.BlockSpec((1, 3, N, bb), lambda i: (i, 0, 0, 0))],
        out_specs=pl.BlockSpec((1, npoint, bb), lambda i: (i, 0, 0)),
        out_shape=jax.ShapeDtypeStruct((halves, npoint, bb), jnp.int32),
        compiler_params=pltpu.CompilerParams(dimension_semantics=("parallel",),
                                             vmem_limit_bytes=48 * 1024 * 1024),
    )(xt)
    return out.transpose(0, 2, 1).reshape(B, npoint)


def _sel_gather_kern(sqdt_ref, trow_ref, src_ref, nxp_ref, ltt_ref, e16_ref,
                     w0_ref, x0_ref, st_ref, *, chunks, chunk, kns, r2):
    srcb = src_ref[0]
    nxb = nxp_ref[0]
    sq = sqdt_ref[0]
    gpc = chunk // kns
    validb = jnp.where(sq <= r2, 1.0, 0.0).astype(jnp.bfloat16)
    rank = jax.lax.dot_general(ltt_ref[...], validb, (((0,), (0,)), ((), ())),
                               preferred_element_type=jnp.float32)
    rv = jnp.where(sq <= r2, jnp.minimum(rank, float(kns + 1)), 0.0)
    rvb = rv.astype(jnp.bfloat16)
    e16 = e16_ref[...]
    w0 = w0_ref[...]
    s1 = jnp.zeros((1, w0.shape[1]), jnp.float32)
    s2 = jnp.zeros((1, w0.shape[1]), jnp.float32)
    for cc in range(chunks):
        sl = slice(gpc * cc, gpc * (cc + 1))
        rvr = jax.lax.dot_general(rvb[:, sl], e16, (((1,), (0,)), ((), ())),
                                  preferred_element_type=jnp.float32)
        tr = trow_ref[0, cc:cc + 1, :]
        oht = jnp.where(rvr == tr, 1.0, 0.0).astype(jnp.bfloat16)
        g = jax.lax.dot_general(oht, srcb, (((0,), (0,)), ((), ())),
                                preferred_element_type=jnp.float32)
        nxg = nxb[sl][:, None, :]
        ctr = jnp.broadcast_to(nxg, (gpc, kns, nxb.shape[-1])).reshape(chunk, -1)
        x0c = (g - ctr).astype(jnp.bfloat16)
        x0_ref[pl.ds(cc * chunk, chunk), :] = x0c
        y0 = jnp.dot(x0c, w0, preferred_element_type=jnp.float32)
        s1 = s1 + jnp.sum(y0, axis=0, keepdims=True)
        s2 = s2 + jnp.sum(y0 * y0, axis=0, keepdims=True)
    st_ref[...] = jnp.concatenate(
        [s1, s2, jnp.zeros((6, s1.shape[1]), jnp.float32)], axis=0)


def _sel_gather(sqdt, trow, src, nxp, w0p, *, nsample):
    B, N, C = src.shape
    S = nxp.shape[1]
    chunks, chunk = trow.shape[1], trow.shape[2]
    rows = chunks * chunk
    c1 = w0p.shape[1]
    ltt = (jnp.arange(N)[:, None] <= jnp.arange(N)[None, :]).astype(jnp.bfloat16)
    e16 = (jnp.arange(chunk // nsample)[:, None]
           == (jnp.arange(chunk)[None, :] // nsample)).astype(jnp.bfloat16)
    r2 = float(0.4 ** 2)
    return pl.pallas_call(
        functools.partial(_sel_gather_kern, chunks=chunks, chunk=chunk,
                          kns=nsample, r2=r2),
        grid=(B,),
        in_specs=[pl.BlockSpec((1, N, S), lambda i: (i, 0, 0)),
                  pl.BlockSpec((1, chunks, chunk), lambda i: (i, 0, 0)),
                  pl.BlockSpec((1, N, C), lambda i: (i, 0, 0)),
                  pl.BlockSpec((1, S, C), lambda i: (i, 0, 0)),
                  pl.BlockSpec((N, N), lambda i: (0, 0)),
                  pl.BlockSpec((chunk // nsample, chunk), lambda i: (0, 0)),
                  pl.BlockSpec((C, c1), lambda i: (0, 0))],
        out_specs=(pl.BlockSpec((rows, C), lambda i: (i, 0)),
                   pl.BlockSpec((8, c1), lambda i: (i, 0))),
        out_shape=(jax.ShapeDtypeStruct((B * rows, C), jnp.bfloat16),
                   jax.ShapeDtypeStruct((B * 8, c1), jnp.float32)),
        compiler_params=pltpu.CompilerParams(dimension_semantics=("parallel",),
                                             vmem_limit_bytes=48 * 1024 * 1024),
    )(sqdt, trow, src, nxp, ltt, e16, w0p)


def _colstats(y):
    s1 = jnp.sum(y, axis=0, keepdims=True)
    s2 = jnp.sum(y * y, axis=0, keepdims=True)
    return jnp.concatenate([s1, s2, jnp.zeros((6, y.shape[1]), jnp.float32)], axis=0)


def _bn_mm_stats_kern(x_ref, w_ref, sc_ref, sh_ref, wn_ref, xn_ref, st_ref):
    y = jnp.dot(x_ref[...], w_ref[...], preferred_element_type=jnp.float32)
    xn = jnp.maximum(y * sc_ref[...] + sh_ref[...], 0.0).astype(jnp.bfloat16)
    xn_ref[...] = xn
    yn = jnp.dot(xn, wn_ref[...], preferred_element_type=jnp.float32)
    st_ref[...] = _colstats(yn)


def _bn_maxpool_kern(x_ref, w_ref, sc_ref, sh_ref, o_ref, *, gpt, nsample):
    y = jnp.dot(x_ref[...], w_ref[...], preferred_element_type=jnp.float32)
    z = jnp.maximum(y * sc_ref[...] + sh_ref[...], 0.0)
    o_ref[...] = jnp.max(z.reshape(gpt, nsample, z.shape[-1]), axis=1)


def _fold_bn(stats, gamma, beta, p, axis_name=None):
    part = stats.reshape(-1, 8, stats.shape[-1]).sum(axis=0)
    if axis_name is not None:
        part = jax.lax.psum(part, axis_name)
    mean = part[0] / float(p)
    var = jnp.maximum(part[1] / float(p) - mean * mean, 0.0)
    sc = gamma.reshape(-1).astype(jnp.float32) * jax.lax.rsqrt(var + _BN_EPS)
    sh = beta.reshape(-1).astype(jnp.float32) - mean * sc
    return sc.reshape(1, -1), sh.reshape(1, -1)


def _mlp_pool(x0, stats0, params, *, nsample, p_total, axis_name=None):
    P, c0 = x0.shape
    (w0, g0, b0), (w1, g1, b1), (w2, g2, b2) = params
    c1, c2, c3 = w0.shape[1], w1.shape[1], w2.shape[1]
    tile = 8192
    nt = P // tile
    gpt = tile // nsample
    cp = pltpu.CompilerParams(dimension_semantics=("parallel",),
                              vmem_limit_bytes=48 * 1024 * 1024)

    sc0, sh0 = _fold_bn(stats0, g0, b0, p_total, axis_name)

    x1, stats1 = pl.pallas_call(
        _bn_mm_stats_kern, grid=(nt,),
        in_specs=[pl.BlockSpec((tile, c0), lambda i: (i, 0)),
                  pl.BlockSpec((c0, c1), lambda i: (0, 0)),
                  pl.BlockSpec((1, c1), lambda i: (0, 0)),
                  pl.BlockSpec((1, c1), lambda i: (0, 0)),
                  pl.BlockSpec((c1, c2), lambda i: (0, 0))],
        out_specs=(pl.BlockSpec((tile, c1), lambda i: (i, 0)),
                   pl.BlockSpec((8, c2), lambda i: (i, 0))),
        out_shape=(jax.ShapeDtypeStruct((P, c1), jnp.bfloat16),
                   jax.ShapeDtypeStruct((nt * 8, c2), jnp.float32)),
        compiler_params=cp,
    )(x0, w0, sc0, sh0, w1)
    sc1, sh1 = _fold_bn(stats1, g1, b1, p_total, axis_name)

    x2, stats2 = pl.pallas_call(
        _bn_mm_stats_kern, grid=(nt,),
        in_specs=[pl.BlockSpec((tile, c1), lambda i: (i, 0)),
                  pl.BlockSpec((c1, c2), lambda i: (0, 0)),
                  pl.BlockSpec((1, c2), lambda i: (0, 0)),
                  pl.BlockSpec((1, c2), lambda i: (0, 0)),
                  pl.BlockSpec((c2, c3), lambda i: (0, 0))],
        out_specs=(pl.BlockSpec((tile, c2), lambda i: (i, 0)),
                   pl.BlockSpec((8, c3), lambda i: (i, 0))),
        out_shape=(jax.ShapeDtypeStruct((P, c2), jnp.bfloat16),
                   jax.ShapeDtypeStruct((nt * 8, c3), jnp.float32)),
        compiler_params=cp,
    )(x1, w1, sc1, sh1, w2)
    sc2, sh2 = _fold_bn(stats2, g2, b2, p_total, axis_name)

    out = pl.pallas_call(
        functools.partial(_bn_maxpool_kern, gpt=gpt, nsample=nsample),
        grid=(nt,),
        in_specs=[pl.BlockSpec((tile, c2), lambda i: (i, 0)),
                  pl.BlockSpec((c2, c3), lambda i: (0, 0)),
                  pl.BlockSpec((1, c3), lambda i: (0, 0)),
                  pl.BlockSpec((1, c3), lambda i: (0, 0))],
        out_specs=pl.BlockSpec((gpt, c3), lambda i: (i, 0)),
        out_shape=jax.ShapeDtypeStruct((P // nsample, c3), jnp.float32),
        compiler_params=cp,
    )(x2, w2, sc2, sh2)
    return out


def _kernel_body(xyz, points, w0, gamma0, beta0, w1, gamma1, beta1,
                 w2, gamma2, beta2, *, p_total, axis_name):
    npoint, radius, nsample = 128, 0.4, 64
    B, _, N = xyz.shape
    xyz_t = jnp.transpose(xyz, (0, 2, 1))
    pts_t = jnp.transpose(points, (0, 2, 1))

    fps_idx = _fps(xyz_t, npoint)
    new_xyz = _onehot_rows(xyz_t, fps_idx)

    sqrdists = _sq_dist(new_xyz, xyz_t)
    sqdt = jnp.transpose(sqrdists, (0, 2, 1))
    cnt = jnp.sum((sqrdists <= radius ** 2).astype(jnp.int32), axis=-1)
    k_iota = jnp.arange(nsample, dtype=jnp.int32)
    trow = jnp.where(k_iota[None, None, :] < cnt[:, :, None], k_iota + 1, 1)
    trow = trow.reshape(B, 8, (npoint * nsample) // 8).astype(jnp.float32)

    c_in = 3 + pts_t.shape[-1]
    c_pad = ((c_in + 15) // 16) * 16
    src = jnp.concatenate(
        [xyz_t, pts_t, jnp.zeros((B, N, c_pad - c_in), jnp.float32)],
        axis=-1).astype(jnp.bfloat16)
    nxp = jnp.concatenate(
        [new_xyz, jnp.zeros((B, npoint, c_pad - 3), jnp.float32)], axis=-1)

    w0p = jnp.pad(w0, ((0, c_pad - c_in), (0, 0))).astype(jnp.bfloat16)
    x0, stats0 = _sel_gather(sqdt, trow, src, nxp, w0p, nsample=nsample)

    params = [(w0p, gamma0, beta0),
              (w1.astype(jnp.bfloat16), gamma1, beta1),
              (w2.astype(jnp.bfloat16), gamma2, beta2)]
    out = _mlp_pool(x0, stats0, params, nsample=nsample,
                    p_total=p_total, axis_name=axis_name)

    feat = out.reshape(B, npoint, -1).transpose(0, 2, 1)
    return jnp.transpose(new_xyz, (0, 2, 1)), feat


def kernel(xyz, points, w0, gamma0, beta0, w1, gamma1, beta1, w2, gamma2, beta2):
    npoint, nsample = 128, 64
    B = xyz.shape[0]
    p_total = B * npoint * nsample
    return _kernel_body(xyz, points, w0, gamma0, beta0, w1, gamma1, beta1,
                        w2, gamma2, beta2, p_total=p_total, axis_name=None)
